# baseline (speedup 1.0000x reference)
import functools
import math

import jax
import jax.numpy as jnp
from jax import lax
from jax.experimental import pallas as pl
from jax.experimental.pallas import tpu as pltpu
from jax.experimental.pallas import tpu_sc as plsc

B = 16384
D = 128
L = 16
NC, NS = 2, 16
NW = NC * NS
ROWS_PER_W = B // NW
R = 128
NCHUNK = ROWS_PER_W // R
NBUF = 2
HALF_LOG_2PI = 0.5 * math.log(2.0 * math.pi)

_mesh = plsc.VectorSubcoreMesh(core_axis_name="c", subcore_axis_name="s")

_scratch = [
    pltpu.VMEM((NBUF, R), jnp.int32),
    pltpu.VMEM((NBUF, R, D), jnp.float32),
    pltpu.VMEM((NBUF, R, D), jnp.float32),
    pltpu.VMEM((NBUF, R), jnp.float32),
    pltpu.VMEM((R * L,), jnp.float32),
    pltpu.SemaphoreType.DMA((NBUF,)),
    pltpu.SemaphoreType.DMA((NBUF,)),
    pltpu.SemaphoreType.DMA((NBUF,)),
    pltpu.SemaphoreType.DMA((NBUF,)),
]


@functools.partial(
    pl.kernel,
    mesh=_mesh,
    compiler_params=pltpu.CompilerParams(needs_layout_passes=False),
    out_type=[
        jax.ShapeDtypeStruct((B,), jnp.float32),
        jax.ShapeDtypeStruct((B, D), jnp.float32),
    ],
    scratch_types=_scratch,
)
def _sc_logprob(x_hbm, y_hbm, ntab_hbm, loss_hbm, lu_hbm,
                idx_v, x_v, o_v, l_v, acc_v, sem_i, sem_x, sem_g, sem_o):
    wid = lax.axis_index("s") * NC + lax.axis_index("c")
    base = wid * ROWS_PER_W
    col_idx = lax.iota(jnp.int32, L) * L

    def issue_in(ci):
        b = lax.rem(ci, NBUF)
        off = base + ci * R
        pltpu.async_copy(y_hbm.at[pl.ds(off, R)], idx_v.at[b], sem_i.at[b])
        pltpu.async_copy(x_hbm.at[pl.ds(off, R), :], x_v.at[b], sem_x.at[b])

    def issue_gather(ci):
        b = lax.rem(ci, NBUF)
        off = base + ci * R
        pltpu.make_async_copy(y_hbm.at[pl.ds(off, R)], idx_v.at[b],
                              sem_i.at[b]).wait()
        pltpu.make_async_copy(x_hbm.at[pl.ds(off, R), :], x_v.at[b],
                              sem_x.at[b]).wait()
        pltpu.async_copy(ntab_hbm.at[idx_v.at[b]], x_v.at[b], sem_g.at[b],
                         add=True)

    def compute(ci):
        b = lax.rem(ci, NBUF)
        pltpu.make_async_copy(ntab_hbm.at[idx_v.at[b]], x_v.at[b],
                              sem_g.at[b]).wait()

        def row_body(row, carry):
            acc = jnp.zeros((L,), jnp.float32)
            for j in range(D // L):
                d = x_v[b, row, pl.ds(j * L, L)]
                o = 0.5 * (d * d) + HALF_LOG_2PI
                acc = acc + o
                o_v[b, row, pl.ds(j * L, L)] = o
            acc_v[pl.ds(row * L, L)] = acc
            return carry

        lax.fori_loop(0, R, row_body, 0, unroll=2)

        def red_body(gi, carry):
            gbase = gi * (L * L) + col_idx
            rowsums = plsc.load_gather(acc_v, [gbase])
            for c in range(1, L):
                rowsums = rowsums + plsc.load_gather(acc_v, [gbase + c])
            l_v[b, pl.ds(gi * L, L)] = rowsums
            return carry

        lax.fori_loop(0, R // L, red_body, 0)

    def issue_out(ci):
        b = lax.rem(ci, NBUF)
        off = base + ci * R
        pltpu.async_copy(o_v.at[b], lu_hbm.at[pl.ds(off, R), :], sem_o.at[b])
        pltpu.async_copy(l_v.at[b], loss_hbm.at[pl.ds(off, R)], sem_o.at[b])

    def wait_out(ci):
        b = lax.rem(ci, NBUF)
        off = base + ci * R
        pltpu.make_async_copy(o_v.at[b], lu_hbm.at[pl.ds(off, R), :],
                              sem_o.at[b]).wait()
        pltpu.make_async_copy(l_v.at[b], loss_hbm.at[pl.ds(off, R)],
                              sem_o.at[b]).wait()

    issue_in(0)
    issue_gather(0)
    issue_in(1)

    def chunk_body(ci, carry):
        @pl.when(ci >= NBUF)
        def _():
            wait_out(ci - NBUF)
        compute(ci)
        issue_out(ci)

        @pl.when(ci + NBUF < NCHUNK)
        def _():
            issue_in(ci + NBUF)

        @pl.when(ci + 1 < NCHUNK)
        def _():
            issue_gather(ci + 1)
        return carry

    lax.fori_loop(0, NCHUNK, chunk_body, 0)
    wait_out(NCHUNK - 2)
    wait_out(NCHUNK - 1)


def kernel(x, y, mu0, mu):
    ntab = -(jnp.concatenate(
        [jnp.zeros((1, D), jnp.float32), mu], axis=0) + mu0[None, :])
    loss, loss_unsummed = _sc_logprob(x, y.astype(jnp.int32), ntab)
    return (loss, loss_unsummed)


# speedup vs baseline: 1.6340x; 1.6340x over previous
import functools
import math

import jax
import jax.numpy as jnp
from jax import lax
from jax.experimental import pallas as pl
from jax.experimental.pallas import tpu as pltpu
from jax.experimental.pallas import tpu_sc as plsc

B = 16384
D = 128
L = 16
NC, NS = 2, 16
NW = NC * NS
ROWS_PER_W = B // NW
R = 128
NCHUNK = ROWS_PER_W // R
NBUF = 3
TPAD = 1024
HALF_LOG_2PI = 0.5 * math.log(2.0 * math.pi)

_mesh = plsc.VectorSubcoreMesh(core_axis_name="c", subcore_axis_name="s")

_scratch = [
    pltpu.VMEM_SHARED((TPAD, D), jnp.float32),
    pltpu.VMEM((NBUF, R), jnp.int32),
    pltpu.VMEM((NBUF, R, D), jnp.float32),
    pltpu.VMEM((NBUF, R, D), jnp.float32),
    pltpu.VMEM((NBUF, R), jnp.float32),
    pltpu.VMEM((L * L,), jnp.float32),
] + [pltpu.SemaphoreType.DMA] * (4 * NBUF + 1)


@functools.partial(
    pl.kernel,
    mesh=_mesh,
    compiler_params=pltpu.CompilerParams(needs_layout_passes=False),
    out_type=[
        jax.ShapeDtypeStruct((B,), jnp.float32),
        jax.ShapeDtypeStruct((B, D), jnp.float32),
    ],
    scratch_types=_scratch,
)
def _sc_logprob(x_hbm, y_hbm, ntab_hbm, loss_hbm, lu_hbm,
                stab, idx_v, x_v, o_v, l_v, acc_v, *sems):
    sem_i = sems[0:NBUF]
    sem_x = sems[NBUF:2 * NBUF]
    sem_g = sems[2 * NBUF:3 * NBUF]
    sem_o = sems[3 * NBUF:4 * NBUF]
    sem_t = sems[4 * NBUF]
    sid = lax.axis_index("s")
    wid = sid * NC + lax.axis_index("c")
    base = wid * ROWS_PER_W
    col_idx = lax.iota(jnp.int32, L) * L

    def issue_in(ci):
        b = ci % NBUF
        off = base + ci * R
        pltpu.async_copy(y_hbm.at[pl.ds(off, R)], idx_v.at[b], sem_i[b])
        pltpu.async_copy(x_hbm.at[pl.ds(off, R), :], x_v.at[b], sem_x[b])

    def issue_gather(ci):
        b = ci % NBUF
        off = base + ci * R
        pltpu.make_async_copy(y_hbm.at[pl.ds(off, R)], idx_v.at[b],
                              sem_i[b]).wait()
        pltpu.make_async_copy(x_hbm.at[pl.ds(off, R), :], x_v.at[b],
                              sem_x[b]).wait()
        pltpu.async_copy(stab.at[idx_v.at[b]], x_v.at[b], sem_g[b],
                         add=True)

    def compute(ci):
        b = ci % NBUF
        pltpu.make_async_copy(stab.at[idx_v.at[b]], x_v.at[b],
                              sem_g[b]).wait()

        def group_body(gi, carry):
            for r16 in range(L):
                row = gi * L + r16
                acc = jnp.zeros((L,), jnp.float32)
                for j in range(D // L):
                    d = x_v[b, row, pl.ds(j * L, L)]
                    o = 0.5 * (d * d) + HALF_LOG_2PI
                    acc = acc + o
                    o_v[b, row, pl.ds(j * L, L)] = o
                acc_v[pl.ds(r16 * L, L)] = acc
            rowsums = jnp.zeros((L,), jnp.float32)
            for c in range(L):
                rowsums = rowsums + plsc.load_gather(acc_v, [col_idx + c])
            l_v[b, pl.ds(gi * L, L)] = rowsums
            return carry

        lax.fori_loop(0, R // L, group_body, 0)

    def issue_out(ci):
        b = ci % NBUF
        off = base + ci * R
        pltpu.async_copy(o_v.at[b], lu_hbm.at[pl.ds(off, R), :], sem_o[b])
        pltpu.async_copy(l_v.at[b], loss_hbm.at[pl.ds(off, R)], sem_o[b])

    def wait_out(ci):
        b = ci % NBUF
        off = base + ci * R
        pltpu.make_async_copy(o_v.at[b], lu_hbm.at[pl.ds(off, R), :],
                              sem_o[b]).wait()
        pltpu.make_async_copy(l_v.at[b], loss_hbm.at[pl.ds(off, R)],
                              sem_o[b]).wait()

    trows = TPAD // NS
    toff = sid * trows
    pltpu.async_copy(ntab_hbm.at[pl.ds(toff, trows), :],
                     stab.at[pl.ds(toff, trows), :], sem_t)
    issue_in(0)
    if NCHUNK > 1:
        issue_in(1)
    pltpu.make_async_copy(ntab_hbm.at[pl.ds(toff, trows), :],
                          stab.at[pl.ds(toff, trows), :], sem_t).wait()
    plsc.subcore_barrier()
    issue_gather(0)

    for ci in range(NCHUNK):
        if ci + 2 < NCHUNK:
            issue_in(ci + 2)
        if ci + 1 < NCHUNK:
            issue_gather(ci + 1)
        if ci >= NBUF:
            wait_out(ci - NBUF)
        compute(ci)
        issue_out(ci)
    for ci in range(max(0, NCHUNK - NBUF), NCHUNK):
        wait_out(ci)


def kernel(x, y, mu0, mu):
    ntab = -(jnp.concatenate(
        [jnp.zeros((1, D), jnp.float32), mu], axis=0) + mu0[None, :])
    ntab_pad = jnp.concatenate(
        [ntab, jnp.zeros((TPAD - ntab.shape[0], D), jnp.float32)], axis=0)
    loss, loss_unsummed = _sc_logprob(x, y.astype(jnp.int32), ntab_pad)
    return (loss, loss_unsummed)
